# baseline (speedup 1.0000x reference)
import jax
import jax.numpy as jnp
from jax import lax
from jax.experimental import pallas as pl
from jax.experimental.pallas import tpu as pltpu
from jax.experimental.pallas import tpu_sc as plsc

B = 16384
LEN = 50
D = 32
NW = 32
BPW = B // NW
HALF_BAGS = 32
NHALF = BPW // HALF_BAGS


def _body(ui_hbm, wi_hbm, l_hbm, U_hbm, W_hbm, out_hbm,
          widx, rows, uidx, urows, lv, outv, sem0, sem1, semu):
    cid = lax.axis_index("c")
    sid = lax.axis_index("s")
    w = sid * 2 + cid
    base = w * BPW

    for i in range(4):
        pltpu.sync_copy(ui_hbm.at[pl.ds(base + i * 128, 128)], uidx.at[i])
    for i in range(4):
        pltpu.async_copy(U_hbm.at[uidx.at[i]], urows.at[pl.ds(i * 128, 128)], semu)
    pltpu.sync_copy(l_hbm.at[pl.ds(base, BPW)], lv)

    sems = (sem0, sem1)

    def fire(h, slot):
        pltpu.sync_copy(wi_hbm.at[pl.ds(base + h * HALF_BAGS, HALF_BAGS)],
                        widx.at[slot])

        def one(i, carry):
            pltpu.async_copy(W_hbm.at[widx.at[slot, i]],
                             rows.at[slot, pl.ds(i * LEN, LEN)], sems[slot])
            return carry
        lax.fori_loop(0, HALF_BAGS, one, 0)

    def drain(slot):
        pltpu.make_async_copy(W_hbm.at[pl.ds(0, HALF_BAGS * LEN)],
                              rows.at[slot], sems[slot]).wait()

    fire(0, 0)
    pltpu.make_async_copy(U_hbm.at[pl.ds(0, BPW)], urows, semu).wait()

    lanes = lax.iota(jnp.int32, 16)
    gdn = lax.GatherDimensionNumbers(
        offset_dims=(), collapsed_slice_dims=(0,), start_index_map=(0,))

    def hsum(t):
        for sh in (8, 4, 2, 1):
            g = lax.gather(t, (lanes ^ sh)[:, None], gdn, slice_sizes=(1,),
                           mode=lax.GatherScatterMode.PROMISE_IN_BOUNDS)
            t = t + g
        return t

    def accum_half(h, slot):
        def bag(i, vec):
            rb = i * LEN
            a00 = rows[slot, rb, pl.ds(0, 16)]
            a01 = rows[slot, rb, pl.ds(16, 16)]
            a10 = rows[slot, rb + 1, pl.ds(0, 16)]
            a11 = rows[slot, rb + 1, pl.ds(16, 16)]
            for j in range(2, LEN, 2):
                a00 = a00 + rows[slot, rb + j, pl.ds(0, 16)]
                a01 = a01 + rows[slot, rb + j, pl.ds(16, 16)]
                a10 = a10 + rows[slot, rb + j + 1, pl.ds(0, 16)]
                a11 = a11 + rows[slot, rb + j + 1, pl.ds(16, 16)]
            s0 = a00 + a10
            s1 = a01 + a11
            gi = h * HALF_BAGS + i
            u0 = urows[gi, pl.ds(0, 16)]
            u1 = urows[gi, pl.ds(16, 16)]
            s = hsum(s0 * u0 + s1 * u1)
            lane = i & 15
            vec = jnp.where(lanes == lane, s, vec)

            @pl.when(lane == 15)
            def _():
                start = pl.multiple_of(gi - 15, 16)
                outv[pl.ds(start, 16)] = vec

            return vec
        lax.fori_loop(0, HALF_BAGS, bag, jnp.zeros((16,), jnp.float32))

    def step(it, carry):
        for b2 in range(2):
            h = it * 2 + b2

            @pl.when(h + 1 < NHALF)
            def _():
                fire(h + 1, 1 - b2)

            drain(b2)
            accum_half(h, b2)
        return carry

    lax.fori_loop(0, NHALF // 2, step, 0)

    def divloop(k, carry):
        sl = pl.ds(k * 16, 16)
        outv[sl] = outv[sl] / lv[sl].astype(jnp.float32)
        return carry

    lax.fori_loop(0, BPW // 16, divloop, 0)
    pltpu.sync_copy(outv, out_hbm.at[pl.ds(base, BPW)])


def kernel(ui, wi, l, U, W):
    mesh = plsc.VectorSubcoreMesh(core_axis_name="c", subcore_axis_name="s")
    run = pl.kernel(
        _body,
        mesh=mesh,
        compiler_params=pltpu.CompilerParams(use_tc_tiling_on_sc=False),
        out_type=jax.ShapeDtypeStruct((B,), jnp.float32),
        scratch_types=[
            pltpu.VMEM((2, HALF_BAGS, LEN), jnp.int32),
            pltpu.VMEM((2, HALF_BAGS * LEN, D), jnp.float32),
            pltpu.VMEM((4, 128), jnp.int32),
            pltpu.VMEM((BPW, D), jnp.float32),
            pltpu.VMEM((BPW,), jnp.int32),
            pltpu.VMEM((BPW,), jnp.float32),
            pltpu.SemaphoreType.DMA,
            pltpu.SemaphoreType.DMA,
            pltpu.SemaphoreType.DMA,
        ],
    )
    return run(ui.astype(jnp.int32), wi.astype(jnp.int32),
               l.astype(jnp.int32), U, W)


# speedup vs baseline: 3.0713x; 3.0713x over previous
import jax
import jax.numpy as jnp
from jax import lax
from jax.experimental import pallas as pl
from jax.experimental.pallas import tpu as pltpu
from jax.experimental.pallas import tpu_sc as plsc

B = 16384
LEN = 50
D = 32
NW = 32
BPW = B // NW
HALF_BAGS = 32
NHALF = BPW // HALF_BAGS


def _body(ui_hbm, wi_hbm, l_hbm, U_hbm, W_hbm, out_hbm,
          widx, rows, uidx, urows, lv, outv, sem0, sem1, semu):
    cid = lax.axis_index("c")
    sid = lax.axis_index("s")
    w = sid * 2 + cid
    base = w * BPW

    for i in range(4):
        pltpu.sync_copy(ui_hbm.at[pl.ds(base + i * 128, 128)], uidx.at[i])
    for i in range(4):
        pltpu.async_copy(U_hbm.at[uidx.at[i]], urows.at[pl.ds(i * 128, 128)], semu)
    pltpu.sync_copy(l_hbm.at[pl.ds(base, BPW)], lv)

    sems = (sem0, sem1)

    def fire(h, slot):
        pltpu.sync_copy(
            wi_hbm.at[pl.ds(w * (BPW // 2) + h * (HALF_BAGS // 2),
                            HALF_BAGS // 2)],
            widx.at[slot])

        def one(i, carry):
            pltpu.async_copy(W_hbm.at[widx.at[slot, i]],
                             rows.at[slot, pl.ds(i * 100, 100)], sems[slot])
            return carry
        lax.fori_loop(0, HALF_BAGS // 2, one, 0)

    def drain(slot):
        pltpu.make_async_copy(W_hbm.at[pl.ds(0, HALF_BAGS * LEN)],
                              rows.at[slot], sems[slot]).wait()

    fire(0, 0)
    pltpu.make_async_copy(U_hbm.at[pl.ds(0, BPW)], urows, semu).wait()

    lanes = lax.iota(jnp.int32, 16)
    gdn = lax.GatherDimensionNumbers(
        offset_dims=(), collapsed_slice_dims=(0,), start_index_map=(0,))

    def hsum(t):
        for sh in (8, 4, 2, 1):
            g = lax.gather(t, (lanes ^ sh)[:, None], gdn, slice_sizes=(1,),
                           mode=lax.GatherScatterMode.PROMISE_IN_BOUNDS)
            t = t + g
        return t

    def accum_half(h, slot):
        def bag(i, vec):
            rb = i * LEN
            a00 = rows[slot, rb, pl.ds(0, 16)]
            a01 = rows[slot, rb, pl.ds(16, 16)]
            a10 = rows[slot, rb + 1, pl.ds(0, 16)]
            a11 = rows[slot, rb + 1, pl.ds(16, 16)]
            for j in range(2, LEN, 2):
                a00 = a00 + rows[slot, rb + j, pl.ds(0, 16)]
                a01 = a01 + rows[slot, rb + j, pl.ds(16, 16)]
                a10 = a10 + rows[slot, rb + j + 1, pl.ds(0, 16)]
                a11 = a11 + rows[slot, rb + j + 1, pl.ds(16, 16)]
            s0 = a00 + a10
            s1 = a01 + a11
            gi = h * HALF_BAGS + i
            u0 = urows[gi, pl.ds(0, 16)]
            u1 = urows[gi, pl.ds(16, 16)]
            s = hsum(s0 * u0 + s1 * u1)
            lane = i & 15
            vec = jnp.where(lanes == lane, s, vec)

            @pl.when(lane == 15)
            def _():
                start = pl.multiple_of(gi - 15, 16)
                outv[pl.ds(start, 16)] = vec

            return vec
        lax.fori_loop(0, HALF_BAGS, bag, jnp.zeros((16,), jnp.float32))

    def step(it, carry):
        for b2 in range(2):
            h = it * 2 + b2

            @pl.when(h + 1 < NHALF)
            def _():
                fire(h + 1, 1 - b2)

            drain(b2)
            accum_half(h, b2)
        return carry

    lax.fori_loop(0, NHALF // 2, step, 0)

    def divloop(k, carry):
        sl = pl.ds(k * 16, 16)
        outv[sl] = outv[sl] / lv[sl].astype(jnp.float32)
        return carry

    lax.fori_loop(0, BPW // 16, divloop, 0)
    pltpu.sync_copy(outv, out_hbm.at[pl.ds(base, BPW)])


BW = 65536
BQ = BW // 4


def _repack_body(wt_ref, out_ref):
    x = wt_ref[...]
    z = jnp.concatenate([x[:, BQ * j:BQ * (j + 1)] for j in range(4)], axis=0)
    out_ref[...] = z.T


def _pidx(r):
    return (r & ~jnp.int32(BW - 1)) | ((r & (BQ - 1)) << 2) | ((r >> 14) & 3)


def _repack(table):
    n = table.shape[0]
    nblk = (n + BW - 1) // BW
    packed = pl.pallas_call(
        _repack_body,
        grid=(nblk,),
        in_specs=[pl.BlockSpec((32, BW), lambda i: (0, i))],
        out_specs=pl.BlockSpec((BQ, 128), lambda i: (i, 0)),
        out_shape=jax.ShapeDtypeStruct((nblk * BQ, 128), jnp.float32),
    )(table.T)
    return packed.reshape(nblk * BW, 32)


def kernel(ui, wi, l, U, W):
    mesh = plsc.VectorSubcoreMesh(core_axis_name="c", subcore_axis_name="s")
    run = pl.kernel(
        _body,
        mesh=mesh,
        compiler_params=pltpu.CompilerParams(use_tc_tiling_on_sc=False),
        out_type=jax.ShapeDtypeStruct((B,), jnp.float32),
        scratch_types=[
            pltpu.VMEM((2, HALF_BAGS // 2, 100), jnp.int32),
            pltpu.VMEM((2, HALF_BAGS * LEN, D), jnp.float32),
            pltpu.VMEM((4, 128), jnp.int32),
            pltpu.VMEM((BPW, D), jnp.float32),
            pltpu.VMEM((BPW,), jnp.int32),
            pltpu.VMEM((BPW,), jnp.float32),
            pltpu.SemaphoreType.DMA,
            pltpu.SemaphoreType.DMA,
            pltpu.SemaphoreType.DMA,
        ],
    )
    wi2 = _pidx(wi.astype(jnp.int32)).reshape(B * LEN // 100, 100)
    return run(_pidx(ui.astype(jnp.int32)), wi2,
               l.astype(jnp.int32), _repack(U), _repack(W))
